# baseline (speedup 1.0000x reference)
import jax
import jax.numpy as jnp
from jax import lax
from jax.experimental import pallas as pl

OUT_D = 1000
B, L = 4096, 20
ROWS = B * L
BLK = 512


def _tc_body(x_ref, o_ref):
    xv = x_ref[...]
    iota = lax.broadcasted_iota(jnp.int32, (BLK, OUT_D), 1)
    o_ref[...] = (iota == xv).astype(jnp.float32)


def kernel(x):
    xf = x.reshape(ROWS, 1)
    out = pl.pallas_call(
        _tc_body,
        grid=(ROWS // BLK,),
        in_specs=[pl.BlockSpec((BLK, 1), lambda i: (i, 0))],
        out_specs=pl.BlockSpec((BLK, OUT_D), lambda i: (i, 0)),
        out_shape=jax.ShapeDtypeStruct((ROWS, OUT_D), jnp.float32),
    )(xf)
    return out.reshape(B, L, OUT_D)


# speedup vs baseline: 1.0785x; 1.0785x over previous
import jax
import jax.numpy as jnp
from jax import lax
from jax.experimental import pallas as pl

OUT_D = 1000
B, L = 4096, 20
ROWS = B * L
BLK = 512


NBLK = ROWS // BLK


def _tc_body(x_ref, o_ref):
    i = pl.program_id(0)
    xv = x_ref[i, 0, :]
    xcol = xv.reshape(BLK, 1)
    iota = lax.broadcasted_iota(jnp.int32, (BLK, OUT_D), 1)
    o_ref[...] = (iota == xcol).astype(jnp.float32)


def kernel(x):
    xf = x.reshape(NBLK, 1, BLK)
    out = pl.pallas_call(
        _tc_body,
        grid=(NBLK,),
        in_specs=[pl.BlockSpec((NBLK, 1, BLK), lambda i: (0, 0, 0))],
        out_specs=pl.BlockSpec((BLK, OUT_D), lambda i: (i, 0)),
        out_shape=jax.ShapeDtypeStruct((ROWS, OUT_D), jnp.float32),
    )(xf)
    return out.reshape(B, L, OUT_D)


# speedup vs baseline: 1.7341x; 1.6079x over previous
import jax
import jax.numpy as jnp
from jax import lax
from jax.experimental import pallas as pl

OUT_D = 1000
B, L = 4096, 20
B0 = 128
NBLK = B // B0


def _body(x_ref, o_ref):
    xb = x_ref[...]
    iota = lax.broadcasted_iota(jnp.int32, (B0, L, OUT_D), 2)
    o_ref[...] = (iota == xb[:, :, None]).astype(jnp.float32)


def kernel(x):
    return pl.pallas_call(
        _body,
        grid=(NBLK,),
        in_specs=[pl.BlockSpec((B0, L), lambda i: (i, 0))],
        out_specs=pl.BlockSpec((B0, L, OUT_D), lambda i: (i, 0, 0)),
        out_shape=jax.ShapeDtypeStruct((B, L, OUT_D), jnp.float32),
    )(x)
